# baseline (speedup 1.0000x reference)
import numpy as np
import jax
import jax.numpy as jnp
from jax.experimental import pallas as pl
from jax.experimental.pallas import tpu as pltpu

_N = 100000
_ROWS = 782
_NP = _ROWS * 128
_K1, _K2, _K3 = 1000, 100, 10
_F = 5
_NEG = float(-np.inf)
_IMAX = 2147483647


def _select_loop(sel_ref, lin, k, gather_fn):
    def body(j, carry):
        sel = sel_ref[...]
        m = jnp.max(sel)
        idx = jnp.min(jnp.where(sel == m, lin, _IMAX))
        sel_ref[...] = jnp.where(lin == idx, _NEG, sel)
        gather_fn(j, idx, m)
        return carry

    jax.lax.fori_loop(0, k, body, 0, unroll=False)


def _net_kernel(xt3_ref, w1_ref, l1wt_ref, l1b_ref, w2_ref,
                w3_ref, l2wt_ref, l2b_ref, out_ref,
                sel1_ref, x1_ref, h_ref, sel2_ref, x2_ref, sel3_ref,
                x3_ref):
    w1 = w1_ref[...]
    norm1 = jnp.sqrt(jnp.sum(w1 * w1))
    s2d = (xt3_ref[0] * w1_ref[0, 0] + xt3_ref[1] * w1_ref[0, 1]
           + xt3_ref[2] * w1_ref[0, 2] + xt3_ref[3] * w1_ref[0, 3]
           + xt3_ref[4] * w1_ref[0, 4])
    r_i = jax.lax.broadcasted_iota(jnp.int32, (_ROWS, 128), 0)
    c_i = jax.lax.broadcasted_iota(jnp.int32, (_ROWS, 128), 1)
    lin1 = r_i * 128 + c_i
    sel1_ref[...] = jnp.where(lin1 < _N, s2d, _NEG)
    lane_i = jax.lax.broadcasted_iota(jnp.int32, (1, 128), 1)

    def gather1(j, idx, m):
        t = jnp.tanh(m / norm1)
        r = idx // 128
        c = idx - r * 128
        for f in range(_F):
            sl = xt3_ref[f, pl.ds(r, 1), :]
            v = jnp.sum(jnp.where(lane_i == c, sl, 0.0))
            x1_ref[pl.ds(j, 1), f:f + 1] = jnp.reshape(v * t, (1, 1))

    _select_loop(sel1_ref, lin1, _K1, gather1)

    x1 = x1_ref[...]
    h = l1b_ref[...] + x1[:, 0:1] * l1wt_ref[0:1, :]
    h = h + x1[:, 1:2] * l1wt_ref[1:2, :]
    h = h + x1[:, 2:3] * l1wt_ref[2:3, :]
    h = h + x1[:, 3:4] * l1wt_ref[3:4, :]
    h = h + x1[:, 4:5] * l1wt_ref[4:5, :]
    h_ref[...] = h

    w2 = w2_ref[...]
    norm2 = jnp.sqrt(jnp.sum(w2 * w2))
    sel2_ref[...] = jnp.sum(h * w2, axis=1, keepdims=True)
    lin2 = jax.lax.broadcasted_iota(jnp.int32, (_K1, 1), 0)

    def gather2(j, idx, m):
        t = jnp.tanh(m / norm2)
        row = h_ref[pl.ds(idx, 1), :]
        x2_ref[pl.ds(j, 1), :] = row * t

    _select_loop(sel2_ref, lin2, _K2, gather2)

    w3 = w3_ref[...]
    norm3 = jnp.sqrt(jnp.sum(w3 * w3))
    x2 = x2_ref[...]
    sel3_ref[...] = jnp.sum(x2 * w3, axis=1, keepdims=True)
    lin3 = jax.lax.broadcasted_iota(jnp.int32, (_K2, 1), 0)

    def gather3(j, idx, m):
        t = jnp.tanh(m / norm3)
        row = x2_ref[pl.ds(idx, 1), :]
        x3_ref[pl.ds(j, 1), :] = row * t

    _select_loop(sel3_ref, lin3, _K3, gather3)

    out = jax.lax.dot_general(
        x3_ref[...], l2wt_ref[...],
        dimension_numbers=(((1,), (0,)), ((), ())),
        preferred_element_type=jnp.float32)
    out_ref[...] = out + l2b_ref[...]


def _run(x, w1, lin1_W, lin1_b, w2, w3, lin2_W, lin2_b):
    xp = jnp.pad(x.astype(jnp.float32), ((0, _NP - _N), (0, 0)))
    xt3 = xp.T.reshape(_F, _ROWS, 128)
    return pl.pallas_call(
        _net_kernel,
        out_shape=jax.ShapeDtypeStruct((_K3, 3), jnp.float32),
        scratch_shapes=[
            pltpu.VMEM((_ROWS, 128), jnp.float32),
            pltpu.VMEM((_K1, _F), jnp.float32),
            pltpu.VMEM((_K1, 64), jnp.float32),
            pltpu.VMEM((_K1, 1), jnp.float32),
            pltpu.VMEM((_K2, 64), jnp.float32),
            pltpu.VMEM((_K2, 1), jnp.float32),
            pltpu.VMEM((_K3, 64), jnp.float32),
        ],
    )(xt3, w1, lin1_W.T, lin1_b.reshape(1, 64), w2, w3,
      lin2_W.T, lin2_b.reshape(1, 3))


_run_jit = jax.jit(_run)


def kernel(x, edge_index, batch, w1, lin1_W, lin1_b, w2, w3, lin2_W, lin2_b):
    return _run_jit(x, w1, lin1_W, lin1_b, w2, w3, lin2_W, lin2_b)
